# Initial kernel scaffold; baseline (speedup 1.0000x reference)
import jax
import jax.numpy as jnp
from jax.experimental import pallas as pl


def kernel(x, W1, b1, g1, bt1, W2, b2, g2, bt2):
    raise NotImplementedError("write your pallas kernel here")


# speedup vs baseline: 5.2972x; 5.2972x over previous
import functools

import jax
import jax.numpy as jnp
from jax import lax
from jax.experimental import pallas as pl
from jax.experimental.pallas import tpu as pltpu
from jax.experimental.pallas import tpu_sc as plsc

K_NBR = 16
GROUPS = 32
EPS = 1e-5

SC_NC = 2
SC_NS = 16
SC_NW = SC_NC * SC_NS

_PREC = lax.Precision.HIGHEST


def _stage_a_body(x_ref, w1_ref, b1_ref, idx_ref, u_ref, v_ref):
    C = x_ref.shape[1]
    N = x_ref.shape[2]
    xb = x_ref[0]
    w1 = w1_ref[...]
    wa = w1[:, :C]
    wb = w1[:, C:]
    inner = lax.dot_general(xb, xb, (((0,), (0,)), ((), ())),
                            preferred_element_type=jnp.float32,
                            precision=lax.Precision.DEFAULT)
    xx = jnp.sum(xb * xb, axis=0)
    dist = xx[:, None] + xx[None, :] - 2.0 * inner
    dist = jnp.maximum(dist, 0.0)
    lane = lax.broadcasted_iota(jnp.int32, (N, N), 1)
    d = dist
    cols = []
    for _ in range(K_NBR):
        m = jnp.min(d, axis=1, keepdims=True)
        am = jnp.min(jnp.where(d == m, lane, N), axis=1, keepdims=True)
        cols.append(am)
        d = jnp.where(lane == am, jnp.inf, d)
    idxb = jnp.concatenate(cols, axis=1)
    idx_ref[0] = idxb + pl.program_id(0) * N
    u_ref[0] = lax.dot_general(xb, wa - wb, (((0,), (1,)), ((), ())),
                               preferred_element_type=jnp.float32,
                               precision=_PREC) + b1_ref[...]
    v_ref[0] = lax.dot_general(xb, wb, (((0,), (1,)), ((), ())),
                               preferred_element_type=jnp.float32,
                               precision=_PREC)


def _stage_a(x, W1, b1):
    B, C, N = x.shape
    return pl.pallas_call(
        _stage_a_body,
        grid=(B,),
        in_specs=[
            pl.BlockSpec((1, C, N), lambda b: (b, 0, 0)),
            pl.BlockSpec((C, 2 * C), lambda b: (0, 0)),
            pl.BlockSpec((1, C), lambda b: (0, 0)),
        ],
        out_specs=[
            pl.BlockSpec((1, N, K_NBR), lambda b: (b, 0, 0)),
            pl.BlockSpec((1, N, C), lambda b: (b, 0, 0)),
            pl.BlockSpec((1, N, C), lambda b: (b, 0, 0)),
        ],
        out_shape=[
            jax.ShapeDtypeStruct((B, N, K_NBR), jnp.int32),
            jax.ShapeDtypeStruct((B, N, C), jnp.float32),
            jax.ShapeDtypeStruct((B, N, C), jnp.float32),
        ],
    )(x, W1, b1)


def _sc_gather(gidx_flat, v_flat):
    E = gidx_flat.shape[0]
    C = v_flat.shape[1]
    epw = E // SC_NW
    ch = 128
    nch = epw // ch
    mesh = plsc.VectorSubcoreMesh(core_axis_name="c", subcore_axis_name="s")

    @functools.partial(
        pl.kernel,
        mesh=mesh,
        out_type=jax.ShapeDtypeStruct((E, C), jnp.float32),
        scratch_types=[
            pltpu.VMEM((ch,), jnp.int32),
            pltpu.VMEM((ch,), jnp.int32),
            pltpu.VMEM((ch, C), jnp.float32),
            pltpu.VMEM((ch, C), jnp.float32),
            pltpu.SemaphoreType.DMA,
            pltpu.SemaphoreType.DMA,
        ],
    )
    def gather_kernel(idx_hbm, tab_hbm, out_hbm, i0, i1, r0, r1, s0, s1):
        wid = lax.axis_index("s") * SC_NC + lax.axis_index("c")
        base = wid * epw
        ibufs = (i0, i1)
        rbufs = (r0, r1)
        sems = (s0, s1)
        cps = [None, None]
        pltpu.sync_copy(idx_hbm.at[pl.ds(base, ch)], i0)
        cps[0] = pltpu.async_copy(tab_hbm.at[i0], r0, s0)
        for t in range(nch):
            nxt = t + 1
            if nxt < nch:
                nb = nxt % 2
                pltpu.sync_copy(idx_hbm.at[pl.ds(base + nxt * ch, ch)],
                                ibufs[nb])
                cps[nb] = pltpu.async_copy(tab_hbm.at[ibufs[nb]], rbufs[nb],
                                           sems[nb])
            cb = t % 2
            cps[cb].wait()
            pltpu.sync_copy(rbufs[cb], out_hbm.at[pl.ds(base + t * ch, ch)])

    return gather_kernel(gidx_flat, v_flat)


def _stage_c_body(g_ref, u_ref, s_ref, q_ref):
    PB = u_ref.shape[1]
    C = u_ref.shape[2]
    g = g_ref[0].reshape(PB, K_NBR, C)
    u = u_ref[0]
    h = g + u[:, None, :]
    s = jnp.sum(h, axis=(0, 1))
    q = jnp.sum(h * h, axis=(0, 1))

    @pl.when(pl.program_id(1) == 0)
    def _():
        s_ref[0, 0, :] = s
        q_ref[0, 0, :] = q

    @pl.when(pl.program_id(1) > 0)
    def _():
        s_ref[0, 0, :] += s
        q_ref[0, 0, :] += q


def _stage_c(G, u, PB):
    B, N, C = u.shape
    NB = N // PB
    return pl.pallas_call(
        _stage_c_body,
        grid=(B, NB),
        in_specs=[
            pl.BlockSpec((1, PB * K_NBR, C), lambda b, nb: (b, nb, 0)),
            pl.BlockSpec((1, PB, C), lambda b, nb: (b, nb, 0)),
        ],
        out_specs=[
            pl.BlockSpec((1, 1, C), lambda b, nb: (b, 0, 0)),
            pl.BlockSpec((1, 1, C), lambda b, nb: (b, 0, 0)),
        ],
        out_shape=[
            jax.ShapeDtypeStruct((B, 1, C), jnp.float32),
            jax.ShapeDtypeStruct((B, 1, C), jnp.float32),
        ],
    )(G, u)


def _group_affine(s_row, q_row, gamma_row, beta_row, count):
    C = s_row.shape[1]
    cpg = C // GROUPS
    r = lax.broadcasted_iota(jnp.int32, (C, C), 0) // cpg
    c = lax.broadcasted_iota(jnp.int32, (C, C), 1) // cpg
    M = (r == c).astype(jnp.float32)
    sg = lax.dot_general(s_row, M, (((1,), (0,)), ((), ())),
                         preferred_element_type=jnp.float32, precision=_PREC)
    qg = lax.dot_general(q_row, M, (((1,), (0,)), ((), ())),
                         preferred_element_type=jnp.float32, precision=_PREC)
    mean = sg / count
    var = qg / count - mean * mean
    scale = gamma_row * lax.rsqrt(var + EPS)
    shift = beta_row - mean * scale
    return scale[0], shift[0]


def _stage_d_body(g_ref, u_ref, s1_ref, q1_ref, g1_ref, bt1_ref, w2_ref,
                  b2_ref, h2_ref, s2_ref, q2_ref, *, count):
    PB = u_ref.shape[1]
    C = u_ref.shape[2]
    scale, shift = _group_affine(s1_ref[0], q1_ref[0], g1_ref[...],
                                 bt1_ref[...], count)
    g = g_ref[0].reshape(PB, K_NBR, C)
    u = u_ref[0]
    h1 = g + u[:, None, :]
    h1n = jnp.maximum(h1 * scale + shift, 0.0).reshape(PB * K_NBR, C)
    h2 = lax.dot_general(h1n, w2_ref[...], (((1,), (1,)), ((), ())),
                         preferred_element_type=jnp.float32,
                         precision=_PREC) + b2_ref[...]
    h2_ref[0] = h2
    s = jnp.sum(h2, axis=0)
    q = jnp.sum(h2 * h2, axis=0)

    @pl.when(pl.program_id(1) == 0)
    def _():
        s2_ref[0, 0, :] = s
        q2_ref[0, 0, :] = q

    @pl.when(pl.program_id(1) > 0)
    def _():
        s2_ref[0, 0, :] += s
        q2_ref[0, 0, :] += q


def _stage_d(G, u, s1, q1, g1, bt1, W2, b2, PB):
    B, N, C = u.shape
    NB = N // PB
    count = float(C // GROUPS) * N * K_NBR
    return pl.pallas_call(
        functools.partial(_stage_d_body, count=count),
        grid=(B, NB),
        in_specs=[
            pl.BlockSpec((1, PB * K_NBR, C), lambda b, nb: (b, nb, 0)),
            pl.BlockSpec((1, PB, C), lambda b, nb: (b, nb, 0)),
            pl.BlockSpec((1, 1, C), lambda b, nb: (b, 0, 0)),
            pl.BlockSpec((1, 1, C), lambda b, nb: (b, 0, 0)),
            pl.BlockSpec((1, C), lambda b, nb: (0, 0)),
            pl.BlockSpec((1, C), lambda b, nb: (0, 0)),
            pl.BlockSpec((C, C), lambda b, nb: (0, 0)),
            pl.BlockSpec((1, C), lambda b, nb: (0, 0)),
        ],
        out_specs=[
            pl.BlockSpec((1, PB * K_NBR, C), lambda b, nb: (b, nb, 0)),
            pl.BlockSpec((1, 1, C), lambda b, nb: (b, 0, 0)),
            pl.BlockSpec((1, 1, C), lambda b, nb: (b, 0, 0)),
        ],
        out_shape=[
            jax.ShapeDtypeStruct((B, N * K_NBR, C), jnp.float32),
            jax.ShapeDtypeStruct((B, 1, C), jnp.float32),
            jax.ShapeDtypeStruct((B, 1, C), jnp.float32),
        ],
    )(G, u, s1, q1, g1, bt1, W2, b2)


def _stage_e_body(h2_ref, s2_ref, q2_ref, g2_ref, bt2_ref, x_ref, y_ref, *,
                  count):
    C = x_ref.shape[1]
    PB = x_ref.shape[2]
    scale, shift = _group_affine(s2_ref[0], q2_ref[0], g2_ref[...],
                                 bt2_ref[...], count)
    h = h2_ref[0].reshape(PB, K_NBR, C)
    hn = jnp.maximum(h * scale + shift, 0.0)
    m = jnp.max(hn, axis=1)
    ri = lax.broadcasted_iota(jnp.int32, (PB, PB), 0)
    ci = lax.broadcasted_iota(jnp.int32, (PB, PB), 1)
    eye = (ri == ci).astype(jnp.float32)
    mt = lax.dot_general(m, eye, (((0,), (0,)), ((), ())),
                         preferred_element_type=jnp.float32, precision=_PREC)
    y_ref[0] = mt + x_ref[0]


def _stage_e(h2, s2, q2, g2, bt2, x, PB):
    B, C, N = x.shape
    NB = N // PB
    count = float(C // GROUPS) * N * K_NBR
    return pl.pallas_call(
        functools.partial(_stage_e_body, count=count),
        grid=(B, NB),
        in_specs=[
            pl.BlockSpec((1, PB * K_NBR, C), lambda b, nb: (b, nb, 0)),
            pl.BlockSpec((1, 1, C), lambda b, nb: (b, 0, 0)),
            pl.BlockSpec((1, 1, C), lambda b, nb: (b, 0, 0)),
            pl.BlockSpec((1, C), lambda b, nb: (0, 0)),
            pl.BlockSpec((1, C), lambda b, nb: (0, 0)),
            pl.BlockSpec((1, C, PB), lambda b, nb: (b, 0, nb)),
        ],
        out_specs=pl.BlockSpec((1, C, PB), lambda b, nb: (b, 0, nb)),
        out_shape=jax.ShapeDtypeStruct((B, C, N), jnp.float32),
    )(h2, s2, q2, g2, bt2, x)


def kernel(x, W1, b1, g1, bt1, W2, b2, g2, bt2):
    B, C, N = x.shape
    PB = 128

    idx, u, v = _stage_a(x, W1, b1.reshape(1, C))
    G = _sc_gather(idx.reshape(B * N * K_NBR), v.reshape(B * N, C))
    G = G.reshape(B, N * K_NBR, C)
    s1, q1 = _stage_c(G, u, PB)
    h2, s2, q2 = _stage_d(G, u, s1, q1, g1.reshape(1, C), bt1.reshape(1, C),
                          W2, b2.reshape(1, C), PB)
    y = _stage_e(h2, s2, q2, g2.reshape(1, C), bt2.reshape(1, C), x, PB)
    return y
